# baseline (speedup 1.0000x reference)
import functools

import jax
import jax.numpy as jnp
from jax import lax
from jax.experimental import pallas as pl
from jax.experimental.pallas import tpu as pltpu
from jax.experimental.pallas import tpu_sc as plsc

_NW = 32
_RW = 8
_NE = 128 // _RW


def _stats_x8(x_rs, pmat):
    R = x_rs.shape[0]

    def body(x_ref, p_ref, st_ref, x8_ref):
        xb = x_ref[...]
        s = jnp.sum(xb, axis=0, keepdims=True)
        q = jnp.sum(xb * xb, axis=0, keepdims=True)
        st_ref[...] = jnp.concatenate(
            [s, q, jnp.zeros((6, 128), jnp.float32)], axis=0)
        x8_ref[...] = jnp.dot(xb, p_ref[...],
                              preferred_element_type=jnp.float32)

    return pl.pallas_call(
        body,
        out_shape=[jax.ShapeDtypeStruct((8, 128), jnp.float32),
                   jax.ShapeDtypeStruct((R, 256), jnp.float32)],
    )(x_rs, pmat)


def _edge_mlp(xi, xj, Akd, Bkd, b1big, W2kd, b2big, W3kd, b3big, crow, *,
              relu_last, br=400):
    ER, L = xi.shape
    if ER % br != 0:
        br = ER
    grid = (ER // br,)
    HB = W2kd.shape[0]

    def body(xi_ref, xj_ref, A_ref, B_ref, b1_ref, W2_ref, b2_ref, W3_ref,
             b3_ref, c_ref, o_ref):
        z = jnp.dot(xi_ref[...], A_ref[...],
                    preferred_element_type=jnp.float32)
        z = z + jnp.dot(xj_ref[...], B_ref[...],
                        preferred_element_type=jnp.float32)
        z = jnp.maximum(z + b1_ref[...], 0.0)
        z = jnp.maximum(
            jnp.dot(z, W2_ref[...], preferred_element_type=jnp.float32)
            + b2_ref[...], 0.0)
        m = jnp.dot(z, W3_ref[...],
                    preferred_element_type=jnp.float32) + b3_ref[...]
        if relu_last:
            m = jnp.maximum(m, 0.0)
        o_ref[...] = m + c_ref[...]

    fixed = lambda shape: pl.BlockSpec(shape, lambda i: (0,) * len(shape))
    return pl.pallas_call(
        body,
        grid=grid,
        in_specs=[
            pl.BlockSpec((br, L), lambda i: (i, 0)),
            pl.BlockSpec((br, L), lambda i: (i, 0)),
            fixed((L, HB)), fixed((L, HB)), fixed((1, HB)),
            fixed((HB, HB)), fixed((1, HB)), fixed((HB, L)),
            fixed((1, L)), fixed((1, L)),
        ],
        out_specs=pl.BlockSpec((br, L), lambda i: (i, 0)),
        out_shape=jax.ShapeDtypeStruct((ER, L), jnp.float32),
    )(xi, xj, Akd, Bkd, b1big, W2kd, b2big, W3kd, b3big, crow)


def _combine_mean(p, sel, m01):
    NR = p.shape[1]

    def body(p_ref, sel_ref, m_ref, h2_ref, c_ref):
        s = p_ref[0] + p_ref[1]
        cnt = jnp.dot(s, sel_ref[...], preferred_element_type=jnp.float32)
        c = jnp.maximum(cnt, 1.0)
        h2_ref[...] = s * m_ref[...] / c
        c_ref[...] = c

    return pl.pallas_call(
        body,
        out_shape=[jax.ShapeDtypeStruct((NR, 128), jnp.float32),
                   jax.ShapeDtypeStruct((NR, 128), jnp.float32)],
    )(p, sel, m01)


def _final_mean(q, c1, u):
    NR = q.shape[1]

    def body(q_ref, c_ref, u_ref, o_ref):
        s = (q_ref[0] + q_ref[1]) / c_ref[...]
        o_ref[...] = jnp.dot(s, u_ref[...],
                             preferred_element_type=jnp.float32)

    return pl.pallas_call(
        body,
        out_shape=jax.ShapeDtypeStruct((NR, 64), jnp.float32),
    )(q, c1, u)


def _gather_rows(table, ei, *, chunk=1000):
    Nn, Dd = table.shape
    E = ei.shape[1]
    assert E % _NW == 0
    epw = E // _NW
    if epw % chunk != 0 or (epw // chunk) % 2 != 0:
        chunk = epw
    assert epw % chunk == 0 and chunk % 8 == 0
    nch = epw // chunk
    mesh = plsc.VectorSubcoreMesh(core_axis_name="c", subcore_axis_name="s",
                                  num_cores=2, num_subcores=16)

    @functools.partial(
        pl.kernel,
        out_type=(jax.ShapeDtypeStruct((E, Dd), jnp.float32),
                  jax.ShapeDtypeStruct((E, Dd), jnp.float32)),
        mesh=mesh,
        compiler_params=pltpu.CompilerParams(use_tc_tiling_on_sc=False),
        scratch_types=[
            [pltpu.VMEM((chunk,), jnp.int32)] * 2,
            [pltpu.VMEM((chunk,), jnp.int32)] * 2,
            [pltpu.VMEM((chunk, Dd), jnp.float32)] * 2,
            [pltpu.VMEM((chunk, Dd), jnp.float32)] * 2,
            [pltpu.SemaphoreType.DMA] * 4,
        ],
    )
    def k(tab, eidx, xi_o, xj_o, idx_d, idx_s, rows_i, rows_j, gsem):
        wid = lax.axis_index("s") * 2 + lax.axis_index("c")
        base = wid * epw

        def load_idx(b, ci):
            off = base + ci * chunk
            pltpu.sync_copy(eidx.at[1, pl.ds(off, chunk)], idx_d[b])
            pltpu.sync_copy(eidx.at[0, pl.ds(off, chunk)], idx_s[b])

        def issue_gather(b):
            pltpu.async_copy(tab.at[idx_d[b]], rows_i[b], gsem[2 * b])
            pltpu.async_copy(tab.at[idx_s[b]], rows_j[b], gsem[2 * b + 1])

        def wait_gather(b):
            pltpu.make_async_copy(tab.at[pl.ds(0, chunk)], rows_i[b],
                                  gsem[2 * b]).wait()
            pltpu.make_async_copy(tab.at[pl.ds(0, chunk)], rows_j[b],
                                  gsem[2 * b + 1]).wait()

        def write_out(b, ci, sems):
            off = base + ci * chunk
            return (pltpu.async_copy(rows_i[b], xi_o.at[pl.ds(off, chunk)],
                                     sems[0]),
                    pltpu.async_copy(rows_j[b], xj_o.at[pl.ds(off, chunk)],
                                     sems[1]))

        def step(b, ci, last):
            wait_gather(b)
            wsems = (gsem[2 * b], gsem[2 * b + 1])
            w1, w2 = write_out(b, ci, wsems)
            if not last:
                load_idx(b, ci + 2)
            w1.wait()
            w2.wait()
            if not last:
                issue_gather(b)

        if nch == 1:
            load_idx(0, 0)
            issue_gather(0)
            step(0, 0, True)
        else:
            load_idx(0, 0)
            issue_gather(0)
            load_idx(1, 1)
            issue_gather(1)

            def pair(p, carry):
                step(0, 2 * p, False)
                step(1, 2 * p + 1, False)
                return carry

            lax.fori_loop(0, nch // 2 - 1, pair, 0)
            step(0, nch - 2, True)
            step(1, nch - 1, True)

    return k(table, ei)


def _scatter_add(m, ei, zeros, n_out, *, chunk=2000):
    E, Dd = m.shape
    NP = zeros.shape[0]
    assert NP % (16 * 8) == 0
    rpt = NP // 16
    last = n_out - 15 * rpt
    assert 0 < last <= rpt and last % 8 == 0
    epw = E // _NW
    if epw % chunk != 0:
        chunk = epw
    assert epw % chunk == 0
    nch = epw // chunk
    mesh = plsc.VectorSubcoreMesh(core_axis_name="c", subcore_axis_name="s",
                                  num_cores=2, num_subcores=16)

    @functools.partial(
        pl.kernel,
        out_type=jax.ShapeDtypeStruct((2, n_out, Dd), jnp.float32),
        mesh=mesh,
        compiler_params=pltpu.CompilerParams(use_tc_tiling_on_sc=False),
        scratch_types=[
            pltpu.VMEM((chunk,), jnp.int32),
            pltpu.VMEM((chunk, Dd), jnp.float32),
            pltpu.VMEM((rpt, Dd), jnp.float32),
            pltpu.VMEM_SHARED((NP, Dd), jnp.float32),
        ],
    )
    def k(mm, eidx, zz, out, idx_v, val_v, row_buf, acc):
        cid = lax.axis_index("c")
        sid = lax.axis_index("s")
        wid = sid * 2 + cid
        rbase = sid * rpt
        pltpu.sync_copy(zz.at[pl.ds(rbase, rpt)], row_buf)
        pltpu.sync_copy(row_buf, acc.at[pl.ds(rbase, rpt)])
        plsc.subcore_barrier()

        def body(ci, carry):
            off = wid * epw + ci * chunk
            pltpu.sync_copy(eidx.at[1, pl.ds(off, chunk)], idx_v)
            pltpu.sync_copy(mm.at[pl.ds(off, chunk)], val_v)
            pltpu.sync_copy(val_v, acc.at[idx_v], add=True)
            return carry

        lax.fori_loop(0, nch, body, 0)
        plsc.subcore_barrier()

        @pl.when(sid < 15)
        def _copy_full():
            pltpu.sync_copy(acc.at[pl.ds(rbase, rpt)], row_buf)
            pltpu.sync_copy(row_buf, out.at[cid, pl.ds(rbase, rpt)])

        @pl.when(sid == 15)
        def _copy_last():
            pltpu.sync_copy(acc.at[pl.ds(rbase, last)],
                            row_buf.at[pl.ds(0, last)])
            pltpu.sync_copy(row_buf.at[pl.ds(0, last)],
                            out.at[cid, pl.ds(rbase, last)])

    return k(m, ei, zeros)


def _pad_cols(a, w):
    r, c = a.shape
    if c == w:
        return a
    return jnp.concatenate([a, jnp.zeros((r, w - c), a.dtype)], axis=1)


def kernel(x, edge_index, gamma, beta, eW1, eb1, eW2, eb2, eW3, eb3,
           dW1, db1, dW2, db2, dW3, db3):
    N, D = x.shape
    E = edge_index.shape[1]
    NP = ((N + 127) // 128) * 128
    NR = N * _RW // 128
    ER = E * _RW // 128
    f32 = jnp.float32
    eyeE = jnp.eye(_NE, dtype=f32)

    def bd(w):
        return jnp.kron(eyeE, w)

    def big(b):
        return jnp.tile(b.reshape(1, -1), (1, _NE))

    e4to8 = jnp.kron(jnp.eye(2 * _NE, dtype=f32),
                     _pad_cols(jnp.eye(D, dtype=f32), _RW))
    selcnt = jnp.kron(eyeE, jnp.zeros((_RW, _RW), f32)
                      .at[eW3.shape[1], :].set(1.0))
    m01 = big(jnp.zeros((_RW,), f32).at[0:2].set(1.0))
    unpack = jnp.kron(eyeE, jnp.eye(_RW, D, dtype=f32))

    stats, x8p = _stats_x8(x.reshape(N * D // 128, 128), e4to8)
    sums = stats[0].reshape(-1, D).sum(axis=0)
    sumsq = stats[1].reshape(-1, D).sum(axis=0)
    mu = sums / N
    var = sumsq / N - mu * mu
    dvec = gamma * lax.rsqrt(var + 1e-5)
    cvec = beta - mu * dvec
    A1 = eW1[:D] - eW1[D:]
    B1 = eW1[D:]
    A1f = _pad_cols((dvec[:, None] * A1).T, _RW).T
    B1f = _pad_cols((dvec[:, None] * B1).T, _RW).T
    b1f = eb1 + cvec @ eW1[:D]

    zeros = jnp.zeros((NP, _RW), f32)

    crow1 = big(jnp.zeros((_RW,), f32).at[eW3.shape[1]].set(1.0))
    xi, xj = _gather_rows(x8p.reshape(N, _RW), edge_index)
    m1 = _edge_mlp(xi.reshape(ER, 128), xj.reshape(ER, 128),
                   bd(A1f), bd(B1f), big(b1f), bd(eW2), big(eb2), bd(_pad_cols(eW3, _RW)),
                   big(_pad_cols(eb3.reshape(1, -1), _RW)), crow1,
                   relu_last=True)
    p1 = _scatter_add(m1.reshape(E, _RW), edge_index, zeros, N)
    h2, c1 = _combine_mean(p1.reshape(2, NR, 128), selcnt, m01)

    H2 = dW1.shape[0] // 2
    A2 = _pad_cols((dW1[:H2] - dW1[H2:]).T, _RW).T
    B2 = _pad_cols(dW1[H2:].T, _RW).T
    crow2 = jnp.zeros((1, 128), f32)
    xi2, xj2 = _gather_rows(h2.reshape(N, _RW), edge_index)
    m2 = _edge_mlp(xi2.reshape(ER, 128), xj2.reshape(ER, 128),
                   bd(A2), bd(B2), big(db1), bd(dW2), big(db2), bd(_pad_cols(dW3, _RW)),
                   big(_pad_cols(db3.reshape(1, -1), _RW)), crow2,
                   relu_last=False)
    q = _scatter_add(m2.reshape(E, _RW), edge_index, zeros, N)
    out = _final_mean(q.reshape(2, NR, 128), c1, unpack)
    return out.reshape(N, D)


# speedup vs baseline: 1.2130x; 1.2130x over previous
import functools

import jax
import jax.numpy as jnp
from jax import lax
from jax.experimental import pallas as pl
from jax.experimental.pallas import tpu as pltpu
from jax.experimental.pallas import tpu_sc as plsc

_NW = 32
_RW = 8
_NE = 128 // _RW


def _stats_x8(x_rs, pmat):
    R = x_rs.shape[0]

    def body(x_ref, p_ref, st_ref, x8_ref):
        xb = x_ref[...]
        s = jnp.sum(xb, axis=0, keepdims=True)
        q = jnp.sum(xb * xb, axis=0, keepdims=True)
        st_ref[...] = jnp.concatenate(
            [s, q, jnp.zeros((6, 128), jnp.float32)], axis=0)
        x8_ref[...] = jnp.dot(xb, p_ref[...],
                              preferred_element_type=jnp.float32)

    return pl.pallas_call(
        body,
        out_shape=[jax.ShapeDtypeStruct((8, 128), jnp.float32),
                   jax.ShapeDtypeStruct((R, 256), jnp.float32)],
    )(x_rs, pmat)


def _edge_mlp(xi, xj, Akd, Bkd, b1big, W2kd, b2big, W3kd, b3big, crow, *,
              relu_last, br=5000):
    ER, L = xi.shape
    if ER % br != 0:
        br = ER
    grid = (ER // br,)
    HB = W2kd.shape[0]

    def body(xi_ref, xj_ref, A_ref, B_ref, b1_ref, W2_ref, b2_ref, W3_ref,
             b3_ref, c_ref, o_ref):
        z = jnp.dot(xi_ref[...], A_ref[...],
                    preferred_element_type=jnp.float32)
        z = z + jnp.dot(xj_ref[...], B_ref[...],
                        preferred_element_type=jnp.float32)
        z = jnp.maximum(z + b1_ref[...], 0.0)
        z = jnp.maximum(
            jnp.dot(z, W2_ref[...], preferred_element_type=jnp.float32)
            + b2_ref[...], 0.0)
        m = jnp.dot(z, W3_ref[...],
                    preferred_element_type=jnp.float32) + b3_ref[...]
        if relu_last:
            m = jnp.maximum(m, 0.0)
        o_ref[...] = m + c_ref[...]

    fixed = lambda shape: pl.BlockSpec(shape, lambda i: (0,) * len(shape))
    return pl.pallas_call(
        body,
        grid=grid,
        in_specs=[
            pl.BlockSpec((br, L), lambda i: (i, 0)),
            pl.BlockSpec((br, L), lambda i: (i, 0)),
            fixed((L, HB)), fixed((L, HB)), fixed((1, HB)),
            fixed((HB, HB)), fixed((1, HB)), fixed((HB, L)),
            fixed((1, L)), fixed((1, L)),
        ],
        out_specs=pl.BlockSpec((br, L), lambda i: (i, 0)),
        out_shape=jax.ShapeDtypeStruct((ER, L), jnp.float32),
    )(xi, xj, Akd, Bkd, b1big, W2kd, b2big, W3kd, b3big, crow)


def _combine_mean(p, sel, m01):
    NR = p.shape[1]

    def body(p_ref, sel_ref, m_ref, h2_ref, c_ref):
        s = p_ref[0] + p_ref[1]
        cnt = jnp.dot(s, sel_ref[...], preferred_element_type=jnp.float32)
        c = jnp.maximum(cnt, 1.0)
        h2_ref[...] = s * m_ref[...] / c
        c_ref[...] = c

    return pl.pallas_call(
        body,
        out_shape=[jax.ShapeDtypeStruct((NR, 128), jnp.float32),
                   jax.ShapeDtypeStruct((NR, 128), jnp.float32)],
    )(p, sel, m01)


def _final_mean(q, c1, u):
    NR = q.shape[1]

    def body(q_ref, c_ref, u_ref, o_ref):
        s = (q_ref[0] + q_ref[1]) / c_ref[...]
        o_ref[...] = jnp.dot(s, u_ref[...],
                             preferred_element_type=jnp.float32)

    return pl.pallas_call(
        body,
        out_shape=jax.ShapeDtypeStruct((NR, 64), jnp.float32),
    )(q, c1, u)


def _gather_rows(table, ei, *, chunk=5000):
    Nn, Dd = table.shape
    E = ei.shape[1]
    assert E % _NW == 0
    epw = E // _NW
    if epw % chunk != 0:
        chunk = epw
    assert epw % chunk == 0 and chunk % 8 == 0
    nch = epw // chunk
    mesh = plsc.VectorSubcoreMesh(core_axis_name="c", subcore_axis_name="s",
                                  num_cores=2, num_subcores=16)

    @functools.partial(
        pl.kernel,
        out_type=(jax.ShapeDtypeStruct((E, Dd), jnp.float32),
                  jax.ShapeDtypeStruct((E, Dd), jnp.float32)),
        mesh=mesh,
        compiler_params=pltpu.CompilerParams(use_tc_tiling_on_sc=False),
        scratch_types=[
            pltpu.VMEM((chunk,), jnp.int32),
            pltpu.VMEM((chunk,), jnp.int32),
            pltpu.VMEM((chunk, Dd), jnp.float32),
            pltpu.VMEM((chunk, Dd), jnp.float32),
            pltpu.SemaphoreType.DMA,
            pltpu.SemaphoreType.DMA,
        ],
    )
    def k(tab, eidx, xi_o, xj_o, idx_d, idx_s, rows_i, rows_j, sem1, sem2):
        wid = lax.axis_index("s") * 2 + lax.axis_index("c")
        base = wid * epw

        def body(ci, carry):
            off = base + ci * chunk
            pltpu.sync_copy(eidx.at[1, pl.ds(off, chunk)], idx_d)
            pltpu.sync_copy(eidx.at[0, pl.ds(off, chunk)], idx_s)
            c1 = pltpu.async_copy(tab.at[idx_d], rows_i, sem1)
            c2 = pltpu.async_copy(tab.at[idx_s], rows_j, sem2)
            c1.wait()
            c2.wait()
            pltpu.sync_copy(rows_i, xi_o.at[pl.ds(off, chunk)])
            pltpu.sync_copy(rows_j, xj_o.at[pl.ds(off, chunk)])
            return carry

        lax.fori_loop(0, nch, body, 0)

    return k(table, ei)


def _scatter_add(m, ei, zeros, n_out, *, chunk=2000):
    E, Dd = m.shape
    NP = zeros.shape[0]
    assert NP % (16 * 8) == 0
    rpt = NP // 16
    last = n_out - 15 * rpt
    assert 0 < last <= rpt and last % 8 == 0
    epw = E // _NW
    if epw % chunk != 0:
        chunk = epw
    assert epw % chunk == 0
    nch = epw // chunk
    mesh = plsc.VectorSubcoreMesh(core_axis_name="c", subcore_axis_name="s",
                                  num_cores=2, num_subcores=16)

    @functools.partial(
        pl.kernel,
        out_type=jax.ShapeDtypeStruct((2, n_out, Dd), jnp.float32),
        mesh=mesh,
        compiler_params=pltpu.CompilerParams(use_tc_tiling_on_sc=False),
        scratch_types=[
            pltpu.VMEM((chunk,), jnp.int32),
            pltpu.VMEM((chunk, Dd), jnp.float32),
            pltpu.VMEM((rpt, Dd), jnp.float32),
            pltpu.VMEM_SHARED((NP, Dd), jnp.float32),
        ],
    )
    def k(mm, eidx, zz, out, idx_v, val_v, row_buf, acc):
        cid = lax.axis_index("c")
        sid = lax.axis_index("s")
        wid = sid * 2 + cid
        rbase = sid * rpt
        pltpu.sync_copy(zz.at[pl.ds(rbase, rpt)], row_buf)
        pltpu.sync_copy(row_buf, acc.at[pl.ds(rbase, rpt)])
        plsc.subcore_barrier()

        def body(ci, carry):
            off = wid * epw + ci * chunk
            pltpu.sync_copy(eidx.at[1, pl.ds(off, chunk)], idx_v)
            pltpu.sync_copy(mm.at[pl.ds(off, chunk)], val_v)
            pltpu.sync_copy(val_v, acc.at[idx_v], add=True)
            return carry

        lax.fori_loop(0, nch, body, 0)
        plsc.subcore_barrier()

        @pl.when(sid < 15)
        def _copy_full():
            pltpu.sync_copy(acc.at[pl.ds(rbase, rpt)], row_buf)
            pltpu.sync_copy(row_buf, out.at[cid, pl.ds(rbase, rpt)])

        @pl.when(sid == 15)
        def _copy_last():
            pltpu.sync_copy(acc.at[pl.ds(rbase, last)],
                            row_buf.at[pl.ds(0, last)])
            pltpu.sync_copy(row_buf.at[pl.ds(0, last)],
                            out.at[cid, pl.ds(rbase, last)])

    return k(m, ei, zeros)


def _pad_cols(a, w):
    r, c = a.shape
    if c == w:
        return a
    return jnp.concatenate([a, jnp.zeros((r, w - c), a.dtype)], axis=1)


def kernel(x, edge_index, gamma, beta, eW1, eb1, eW2, eb2, eW3, eb3,
           dW1, db1, dW2, db2, dW3, db3):
    N, D = x.shape
    E = edge_index.shape[1]
    NP = ((N + 127) // 128) * 128
    NR = N * _RW // 128
    ER = E * _RW // 128
    f32 = jnp.float32
    eyeE = jnp.eye(_NE, dtype=f32)

    def bd(w):
        return jnp.kron(eyeE, w)

    def big(b):
        return jnp.tile(b.reshape(1, -1), (1, _NE))

    e4to8 = jnp.kron(jnp.eye(2 * _NE, dtype=f32),
                     _pad_cols(jnp.eye(D, dtype=f32), _RW))
    selcnt = jnp.kron(eyeE, jnp.zeros((_RW, _RW), f32)
                      .at[eW3.shape[1], :].set(1.0))
    m01 = big(jnp.zeros((_RW,), f32).at[0:2].set(1.0))
    unpack = jnp.kron(eyeE, jnp.eye(_RW, D, dtype=f32))

    stats, x8p = _stats_x8(x.reshape(N * D // 128, 128), e4to8)
    sums = stats[0].reshape(-1, D).sum(axis=0)
    sumsq = stats[1].reshape(-1, D).sum(axis=0)
    mu = sums / N
    var = sumsq / N - mu * mu
    dvec = gamma * lax.rsqrt(var + 1e-5)
    cvec = beta - mu * dvec
    A1 = eW1[:D] - eW1[D:]
    B1 = eW1[D:]
    A1f = _pad_cols((dvec[:, None] * A1).T, _RW).T
    B1f = _pad_cols((dvec[:, None] * B1).T, _RW).T
    b1f = eb1 + cvec @ eW1[:D]

    zeros = jnp.zeros((NP, _RW), f32)

    crow1 = big(jnp.zeros((_RW,), f32).at[eW3.shape[1]].set(1.0))
    xi, xj = _gather_rows(x8p.reshape(N, _RW), edge_index)
    m1 = _edge_mlp(xi.reshape(ER, 128), xj.reshape(ER, 128),
                   bd(A1f), bd(B1f), big(b1f), bd(eW2), big(eb2), bd(_pad_cols(eW3, _RW)),
                   big(_pad_cols(eb3.reshape(1, -1), _RW)), crow1,
                   relu_last=True)
    p1 = _scatter_add(m1.reshape(E, _RW), edge_index, zeros, N)
    h2, c1 = _combine_mean(p1.reshape(2, NR, 128), selcnt, m01)

    H2 = dW1.shape[0] // 2
    A2 = _pad_cols((dW1[:H2] - dW1[H2:]).T, _RW).T
    B2 = _pad_cols(dW1[H2:].T, _RW).T
    crow2 = jnp.zeros((1, 128), f32)
    xi2, xj2 = _gather_rows(h2.reshape(N, _RW), edge_index)
    m2 = _edge_mlp(xi2.reshape(ER, 128), xj2.reshape(ER, 128),
                   bd(A2), bd(B2), big(db1), bd(dW2), big(db2), bd(_pad_cols(dW3, _RW)),
                   big(_pad_cols(db3.reshape(1, -1), _RW)), crow2,
                   relu_last=False)
    q = _scatter_add(m2.reshape(E, _RW), edge_index, zeros, N)
    out = _final_mean(q.reshape(2, NR, 128), c1, unpack)
    return out.reshape(N, D)
